# baseline (speedup 1.0000x reference)
import jax
import jax.numpy as jnp
from jax import lax
from jax.experimental import pallas as pl
from jax.experimental.pallas import tpu as pltpu
from jax.experimental.pallas import tpu_sc as plsc

EMBED_DIM = 128
NUM_CORES = 2
NUM_SUBCORES = 16
NUM_WORKERS = NUM_CORES * NUM_SUBCORES

CHUNK_ROWS = 128
IDX_BLOCK = 32
NBUF = 2


def _embed_body(x2d_hbm, table_hbm, out_hbm, idx_v, rows_v, sem_g, sem_o,
                sem_i, table_sh):
  n_rows = out_hbm.shape[0]
  rows_per_worker = n_rows // NUM_WORKERS
  chunks_per_worker = rows_per_worker // CHUNK_ROWS
  n_idx_blocks = chunks_per_worker // IDX_BLOCK

  sid = lax.axis_index("s")
  wid = sid * NUM_CORES + lax.axis_index("c")
  chunk0 = wid * chunks_per_worker

  pltpu.sync_copy(table_hbm, table_sh.at[sid])
  tab = table_sh.at[sid]

  def idx_block_copy(i, parity):
    iblk = pl.multiple_of(chunk0 + i * IDX_BLOCK, IDX_BLOCK)
    return pltpu.make_async_copy(
        x2d_hbm.at[pl.ds(iblk, IDX_BLOCK)], idx_v.at[parity], sem_i.at[parity])

  def drain(b, c_prev):
    row0 = pl.multiple_of(c_prev * CHUNK_ROWS, CHUNK_ROWS)
    pltpu.make_async_copy(
        tab.at[idx_v.at[0].at[0]], rows_v.at[b], sem_g.at[b]).wait()
    pltpu.async_copy(
        rows_v.at[b], out_hbm.at[pl.ds(row0, CHUNK_ROWS)], sem_o.at[b]).wait()

  idx_block_copy(0, 0).start()

  def outer(i, _):
    p = lax.rem(i, 2)
    idx_block_copy(i, p).wait()

    def inner(jj, _):
      for b in range(NBUF):
        j = jj * NBUF + b
        g = i * IDX_BLOCK + j
        c = chunk0 + g

        @pl.when(g >= NBUF)
        def _():
          drain(b, c - NBUF)

        pltpu.async_copy(
            tab.at[idx_v.at[p].at[j]], rows_v.at[b], sem_g.at[b])
      return ()

    inner(0, ())

    @pl.when(i + 1 < n_idx_blocks)
    def _():
      idx_block_copy(i + 1, 1 - p).start()

    lax.fori_loop(1, IDX_BLOCK // NBUF, inner, (), unroll=False)
    return ()

  lax.fori_loop(0, n_idx_blocks, outer, (), unroll=False)

  n_chunks = chunks_per_worker
  for t in range(NBUF):
    g = n_chunks - NBUF + t
    drain(g % NBUF, chunk0 + g)


@jax.jit
def kernel(x, table):
  batch, seq = x.shape
  n_rows = batch * seq
  x2d = x.reshape(n_rows // EMBED_DIM, EMBED_DIM).astype(jnp.int32)
  table = table.astype(jnp.float32)

  mesh = plsc.VectorSubcoreMesh(core_axis_name="c", subcore_axis_name="s")
  run = pl.kernel(
      _embed_body,
      out_type=jax.ShapeDtypeStruct((n_rows, EMBED_DIM), jnp.float32),
      mesh=mesh,
      scratch_types=[
          pltpu.VMEM((2, IDX_BLOCK, EMBED_DIM), jnp.int32),
          pltpu.VMEM((NBUF, CHUNK_ROWS, EMBED_DIM), jnp.float32),
          pltpu.SemaphoreType.DMA((NBUF,)),
          pltpu.SemaphoreType.DMA((NBUF,)),
          pltpu.SemaphoreType.DMA((2,)),
          pltpu.VMEM_SHARED((NUM_SUBCORES, 4, EMBED_DIM), jnp.float32),
      ],
  )
  out = run(x2d, table)
  return out.reshape(batch, seq, EMBED_DIM)


# speedup vs baseline: 1.0212x; 1.0212x over previous
import jax
import jax.numpy as jnp
from jax import lax
from jax.experimental import pallas as pl
from jax.experimental.pallas import tpu as pltpu
from jax.experimental.pallas import tpu_sc as plsc

EMBED_DIM = 128
NUM_CORES = 2
NUM_SUBCORES = 16
NUM_WORKERS = NUM_CORES * NUM_SUBCORES

CHUNK_ROWS = 128
IDX_BLOCK = 32
NBUF = 4


def _embed_body(x2d_hbm, table_hbm, out_hbm, idx_v, rows_v, sem_g, sem_o,
                sem_i, table_sh):
  n_rows = out_hbm.shape[0]
  rows_per_worker = n_rows // NUM_WORKERS
  chunks_per_worker = rows_per_worker // CHUNK_ROWS
  n_idx_blocks = chunks_per_worker // IDX_BLOCK

  sid = lax.axis_index("s")
  wid = sid * NUM_CORES + lax.axis_index("c")
  chunk0 = wid * chunks_per_worker

  pltpu.sync_copy(table_hbm, table_sh.at[sid])
  tab = table_sh.at[sid]

  def idx_block_copy(i, parity):
    iblk = pl.multiple_of(chunk0 + i * IDX_BLOCK, IDX_BLOCK)
    return pltpu.make_async_copy(
        x2d_hbm.at[pl.ds(iblk, IDX_BLOCK)], idx_v.at[parity], sem_i.at[parity])

  def drain(b, c_prev):
    row0 = pl.multiple_of(c_prev * CHUNK_ROWS, CHUNK_ROWS)
    pltpu.make_async_copy(
        tab.at[idx_v.at[0].at[0]], rows_v.at[b], sem_g.at[b]).wait()
    pltpu.async_copy(
        rows_v.at[b], out_hbm.at[pl.ds(row0, CHUNK_ROWS)], sem_o.at[b]).wait()

  idx_block_copy(0, 0).start()

  def outer(i, _):
    p = lax.rem(i, 2)
    idx_block_copy(i, p).wait()

    def inner(jj, _):
      for b in range(NBUF):
        j = jj * NBUF + b
        g = i * IDX_BLOCK + j
        c = chunk0 + g

        @pl.when(g >= NBUF)
        def _():
          drain(b, c - NBUF)

        pltpu.async_copy(
            tab.at[idx_v.at[p].at[j]], rows_v.at[b], sem_g.at[b])
      return ()

    inner(0, ())

    @pl.when(i + 1 < n_idx_blocks)
    def _():
      idx_block_copy(i + 1, 1 - p).start()

    lax.fori_loop(1, IDX_BLOCK // NBUF, inner, (), unroll=False)
    return ()

  lax.fori_loop(0, n_idx_blocks, outer, (), unroll=False)

  n_chunks = chunks_per_worker
  for t in range(NBUF):
    g = n_chunks - NBUF + t
    drain(g % NBUF, chunk0 + g)


@jax.jit
def kernel(x, table):
  batch, seq = x.shape
  n_rows = batch * seq
  x2d = x.reshape(n_rows // EMBED_DIM, EMBED_DIM).astype(jnp.int32)
  table = table.astype(jnp.float32)

  mesh = plsc.VectorSubcoreMesh(core_axis_name="c", subcore_axis_name="s")
  run = pl.kernel(
      _embed_body,
      out_type=jax.ShapeDtypeStruct((n_rows, EMBED_DIM), jnp.float32),
      mesh=mesh,
      scratch_types=[
          pltpu.VMEM((2, IDX_BLOCK, EMBED_DIM), jnp.int32),
          pltpu.VMEM((NBUF, CHUNK_ROWS, EMBED_DIM), jnp.float32),
          pltpu.SemaphoreType.DMA((NBUF,)),
          pltpu.SemaphoreType.DMA((NBUF,)),
          pltpu.SemaphoreType.DMA((2,)),
          pltpu.VMEM_SHARED((NUM_SUBCORES, 4, EMBED_DIM), jnp.float32),
      ],
  )
  out = run(x2d, table)
  return out.reshape(batch, seq, EMBED_DIM)
